# baseline (speedup 1.0000x reference)
import functools

import jax
import jax.numpy as jnp
from jax import lax
from jax.experimental import pallas as pl
from jax.experimental.pallas import tpu as pltpu
from jax.experimental.pallas import tpu_sc as plsc

N = 10000
E = 320000
D = 128
DE = 16
EH = 128
OE = 16
NH = 128
ON = 128

NC = 2
NS = 16
NW = NC * NS

WROWS = 81
EPAD = NW * WROWS * 128

NPAD = 10240
STRIPE = NPAD // NS

BE = 4096
BN = 2000

_SQRT_HALF = 0.7071067811865476


def _ln_gelu(x, g, b):
    mu = jnp.mean(x, axis=-1, keepdims=True)
    xc = x - mu
    var = jnp.mean(xc * xc, axis=-1, keepdims=True)
    y = xc * lax.rsqrt(var + 1e-5) * g + b
    return 0.5 * y * (1.0 + lax.erf(y * _SQRT_HALF))


C0ROWS = 123
C1ROWS = 39


def _sc_gather(x, row2d, col2d):
    mesh = plsc.VectorSubcoreMesh(core_axis_name="c", subcore_axis_name="s", num_cores=NC, num_subcores=NS)

    @functools.partial(
        pl.kernel,
        out_type=[
            jax.ShapeDtypeStruct((EPAD, D), jnp.float32),
            jax.ShapeDtypeStruct((EPAD, D), jnp.float32),
        ],
        mesh=mesh,
        scratch_types=[
            pltpu.VMEM((C0ROWS, 128), jnp.int32),
            pltpu.VMEM((C0ROWS, 128), jnp.int32),
            pltpu.VMEM((128, D), jnp.float32),
            pltpu.VMEM((128, D), jnp.float32),
            pltpu.SemaphoreType.DMA,
            pltpu.SemaphoreType.DMA,
        ],
        compiler_params=pltpu.CompilerParams(use_tc_tiling_on_sc=False),
    )
    def gather_k(x_hbm, row_hbm, col_hbm, gs_hbm, gt_hbm,
                 ridx_v, cidx_v, s_v, t_v, sem_s, sem_t):
        cid = lax.axis_index("c")
        sid = lax.axis_index("s")
        nrows = jnp.where(cid == 0, C0ROWS, C1ROWS)
        base = sid * (C0ROWS + C1ROWS) + jnp.where(cid == 0, 0, C0ROWS)

        pltpu.sync_copy(row_hbm.at[pl.ds(base, C1ROWS)],
                        ridx_v.at[pl.ds(0, C1ROWS)])
        pltpu.sync_copy(col_hbm.at[pl.ds(base, C1ROWS)],
                        cidx_v.at[pl.ds(0, C1ROWS)])

        @pl.when(cid == 0)
        def _():
            pltpu.sync_copy(row_hbm.at[pl.ds(base + C1ROWS, C0ROWS - C1ROWS)],
                            ridx_v.at[pl.ds(C1ROWS, C0ROWS - C1ROWS)])
            pltpu.sync_copy(col_hbm.at[pl.ds(base + C1ROWS, C0ROWS - C1ROWS)],
                            cidx_v.at[pl.ds(C1ROWS, C0ROWS - C1ROWS)])

        def body(j, carry):
            off = (base + j) * 128
            cps = pltpu.async_copy(x_hbm.at[ridx_v.at[j]], s_v, sem_s)
            cpt = pltpu.async_copy(x_hbm.at[cidx_v.at[j]], t_v, sem_t)
            cps.wait()
            pltpu.sync_copy(s_v, gs_hbm.at[pl.ds(off, 128)])
            cpt.wait()
            pltpu.sync_copy(t_v, gt_hbm.at[pl.ds(off, 128)])
            return carry

        lax.fori_loop(0, nrows, body, 0)

    return gather_k(x, row2d, col2d)


def _sc_segment_sum(eout_p, col2d):
    mesh = plsc.VectorSubcoreMesh(core_axis_name="c", subcore_axis_name="s", num_cores=NC, num_subcores=NS)
    CR = 9

    @functools.partial(
        pl.kernel,
        out_type=jax.ShapeDtypeStruct((NC, NPAD, OE), jnp.float32),
        mesh=mesh,
        scratch_types=[
            pltpu.VMEM((CR, 128), jnp.int32),
            pltpu.VMEM((CR * 16, 128), jnp.float32),
            pltpu.VMEM((CR * 128, OE), jnp.float32),
            pltpu.VMEM((STRIPE, OE), jnp.float32),
            pltpu.VMEM_SHARED((NPAD, OE), jnp.float32),
        ],
        compiler_params=pltpu.CompilerParams(use_tc_tiling_on_sc=False),
    )
    def scatter_k(eout_pk_hbm, col_hbm, out_hbm, idx_v, rows_v, rows_e, zb_v,
                  acc_sp):
        cid = lax.axis_index("c")
        sid = lax.axis_index("s")
        wid = sid * NC + cid

        def zbody(i, carry):
            zb_v[i] = jnp.zeros((OE,), jnp.float32)
            return carry

        lax.fori_loop(0, STRIPE, zbody, 0)
        pltpu.sync_copy(zb_v, acc_sp.at[pl.ds(sid * STRIPE, STRIPE)])
        plsc.subcore_barrier()

        base_row = wid * WROWS

        def body(j, carry):
            r0 = base_row + j * CR
            pltpu.sync_copy(col_hbm.at[pl.ds(r0, CR)], idx_v)
            pltpu.sync_copy(eout_pk_hbm.at[pl.ds(r0 * 16, CR * 16)], rows_v)

            def unpack(r, c3):
                for k in range(8):
                    rows_e[r * 8 + k] = rows_v[r, pl.ds(16 * k, 16)]
                return c3

            lax.fori_loop(0, CR * 16, unpack, 0)

            def inner(k, c2):
                pltpu.sync_copy(rows_e.at[pl.ds(k * 128, 128)],
                                acc_sp.at[idx_v.at[k]], add=True)
                return c2

            lax.fori_loop(0, CR, inner, 0)
            return carry

        lax.fori_loop(0, WROWS // CR, body, 0)
        plsc.subcore_barrier()

        pltpu.sync_copy(acc_sp.at[pl.ds(sid * STRIPE, STRIPE)], zb_v)
        pltpu.sync_copy(zb_v, out_hbm.at[cid].at[pl.ds(sid * STRIPE, STRIPE)])

    return scatter_k(eout_p, col2d)


def _edge_body(gs_ref, gt_ref, ef_ref, A_ref, Bm_ref, Wef_ref, wsq_ref,
               wdist_ref, eb1_ref, eg1_ref, ebt1_ref, eW2_ref, eb2_ref,
               eg2_ref, ebt2_ref, out_ref):
    s = gs_ref[...]
    t = gt_ref[...]
    efp = ef_ref[...]
    ef = jnp.stack([efp[:, 16 * k:16 * (k + 1)] for k in range(8)],
                   axis=1).reshape(BE, DE)
    diff = s - t
    sq = jnp.sum(diff * diff, axis=1, keepdims=True)
    dist = jnp.sqrt(sq + 1e-12)
    h = (jnp.dot(s, A_ref[...], preferred_element_type=jnp.float32)
         + jnp.dot(t, Bm_ref[...], preferred_element_type=jnp.float32)
         + jnp.dot(ef, Wef_ref[...], preferred_element_type=jnp.float32)
         + sq * wsq_ref[...] + dist * wdist_ref[...] + eb1_ref[...])
    h = _ln_gelu(h, eg1_ref[...], ebt1_ref[...])
    h2 = jnp.dot(h, eW2_ref[...], preferred_element_type=jnp.float32) + eb2_ref[...]
    h2 = _ln_gelu(h2, eg2_ref[...], ebt2_ref[...])
    gid = pl.program_id(0) * BE + lax.broadcasted_iota(jnp.int32, (BE, 1), 0)
    h2 = jnp.where(gid < E, h2, 0.0)
    h3 = h2.reshape(BE // 8, 8, OE)
    out_ref[...] = jnp.concatenate([h3[:, k, :] for k in range(8)], axis=1)


def _tc_edge(gs, gt, ef_p, A, Bm, Wef, wsq, wdist, eb1, eg1, ebt1,
             eW2, eb2, eg2, ebt2):
    full = lambda shape: pl.BlockSpec(shape, lambda i: (0, 0))
    return pl.pallas_call(
        _edge_body,
        grid=(EPAD // BE,),
        in_specs=[
            pl.BlockSpec((BE, D), lambda i: (i, 0)),
            pl.BlockSpec((BE, D), lambda i: (i, 0)),
            pl.BlockSpec((BE // 8, 128), lambda i: (i, 0)),
            full((D, EH)), full((D, EH)), full((DE, EH)),
            full((1, EH)), full((1, EH)), full((1, EH)), full((1, EH)),
            full((1, EH)),
            full((EH, OE)), full((1, OE)), full((1, OE)), full((1, OE)),
        ],
        out_specs=pl.BlockSpec((BE // 8, 128), lambda i: (i, 0)),
        out_shape=jax.ShapeDtypeStruct((EPAD // 8, 128), jnp.float32),
    )(gs, gt, ef_p, A, Bm, Wef, wsq, wdist, eb1, eg1, ebt1,
      eW2, eb2, eg2, ebt2)


def _node_body(x_ref, ap_ref, W1x_ref, W1a_ref, nb1_ref, ng1_ref, nbt1_ref,
               nW2_ref, nb2_ref, ng2_ref, nbt2_ref, out_ref):
    x = x_ref[...]
    a = ap_ref[0] + ap_ref[1]
    h = (jnp.dot(x, W1x_ref[...], preferred_element_type=jnp.float32)
         + jnp.dot(a, W1a_ref[...], preferred_element_type=jnp.float32)
         + nb1_ref[...])
    h = _ln_gelu(h, ng1_ref[...], nbt1_ref[...])
    o = jnp.dot(h, nW2_ref[...], preferred_element_type=jnp.float32) + nb2_ref[...]
    out_ref[...] = _ln_gelu(o, ng2_ref[...], nbt2_ref[...])


def _tc_node(x, ap, W1x, W1a, nb1, ng1, nbt1, nW2, nb2, ng2, nbt2):
    full = lambda shape: pl.BlockSpec(shape, lambda i: (0, 0))
    full3 = lambda shape: pl.BlockSpec(shape, lambda i: (0, i, 0))
    return pl.pallas_call(
        _node_body,
        grid=(N // BN,),
        in_specs=[
            pl.BlockSpec((BN, D), lambda i: (i, 0)),
            full3((NC, BN, OE)),
            full((D, NH)), full((OE, NH)),
            full((1, NH)), full((1, NH)), full((1, NH)),
            full((NH, ON)), full((1, ON)), full((1, ON)), full((1, ON)),
        ],
        out_specs=pl.BlockSpec((BN, ON), lambda i: (i, 0)),
        out_shape=jax.ShapeDtypeStruct((N, ON), jnp.float32),
    )(x, ap, W1x, W1a, nb1, ng1, nbt1, nW2, nb2, ng2, nbt2)


def kernel(node_features, edge_index, edge_features,
           eW1, eb1, eg1, ebt1, eW2, eb2, eg2, ebt2,
           nW1, nb1, ng1, nbt1, nW2, nb2, ng2, nbt2):
    row = edge_index[0].astype(jnp.int32)
    col = edge_index[1].astype(jnp.int32)
    row2d = jnp.pad(row, (0, EPAD - E)).reshape(EPAD // 128, 128)
    col2d = jnp.pad(col, (0, EPAD - E)).reshape(EPAD // 128, 128)
    ef_p = jnp.pad(edge_features, ((0, EPAD - E), (0, 0))).reshape(EPAD // 8, 128)

    A = eW1[0:D] + eW1[2 * D:3 * D]
    Bm = eW1[D:2 * D] - eW1[2 * D:3 * D]
    wsq = eW1[3 * D:3 * D + 1]
    wdist = eW1[3 * D + 1:3 * D + 2]
    Wef = eW1[3 * D + 2:]

    r2 = lambda v: v.reshape(1, -1)

    gs, gt = _sc_gather(node_features, row2d, col2d)
    eout_p = _tc_edge(gs, gt, ef_p, A, Bm, Wef, wsq, wdist,
                      r2(eb1), r2(eg1), r2(ebt1), eW2, r2(eb2), r2(eg2),
                      r2(ebt2))
    e_out = eout_p.reshape(EPAD, OE)[:E]

    partials = _sc_segment_sum(eout_p, col2d)

    W1x = nW1[0:D]
    W1a = nW1[D:]
    n_out = _tc_node(node_features, partials, W1x, W1a,
                     r2(nb1), r2(ng1), r2(nbt1),
                     nW2, r2(nb2), r2(ng2), r2(nbt2))
    return (n_out, e_out)


# speedup vs baseline: 1.2340x; 1.2340x over previous
import functools

import jax
import jax.numpy as jnp
from jax import lax
from jax.experimental import pallas as pl
from jax.experimental.pallas import tpu as pltpu
from jax.experimental.pallas import tpu_sc as plsc

N = 10000
E = 320000
D = 128
DE = 16
EH = 128
OE = 16
NH = 128
ON = 128

NC = 2
NS = 16
NW = NC * NS

WROWS = 80
EPAD = NW * WROWS * 128

NPAD = 10240
STRIPE = NPAD // NS

BE = 4096
BN = 2000

_SQRT_HALF = 0.7071067811865476


def _ln_gelu(x, g, b):
    mu = jnp.mean(x, axis=-1, keepdims=True)
    xc = x - mu
    var = jnp.mean(xc * xc, axis=-1, keepdims=True)
    y = xc * lax.rsqrt(var + 1e-5) * g + b
    return 0.5 * y * (1.0 + lax.erf(y * _SQRT_HALF))


def _sc_gather(x, row_p, col_p):
    mesh = plsc.VectorSubcoreMesh(core_axis_name="c", subcore_axis_name="s",
                                  num_cores=NC, num_subcores=NS)

    @functools.partial(
        pl.kernel,
        out_type=[
            jax.ShapeDtypeStruct((EPAD, D), jnp.float32),
            jax.ShapeDtypeStruct((EPAD, D), jnp.float32),
        ],
        mesh=mesh,
        scratch_types=[
            pltpu.VMEM((128,), jnp.int32),
            pltpu.VMEM((128,), jnp.int32),
            pltpu.VMEM((128, D), jnp.float32),
            pltpu.VMEM((128, D), jnp.float32),
            pltpu.SemaphoreType.DMA,
            pltpu.SemaphoreType.DMA,
        ],
        compiler_params=pltpu.CompilerParams(use_tc_tiling_on_sc=False),
    )
    def gather_k(x_hbm, row_hbm, col_hbm, gs_hbm, gt_hbm,
                 idxr_v, idxc_v, s_v, t_v, sem_s, sem_t):
        wid = lax.axis_index("s") * NC + lax.axis_index("c")
        base = wid * (WROWS * 128)

        def body(j, carry):
            off = base + j * 128
            pltpu.sync_copy(row_hbm.at[pl.ds(off, 128)], idxr_v)
            pltpu.sync_copy(col_hbm.at[pl.ds(off, 128)], idxc_v)
            cps = pltpu.async_copy(x_hbm.at[idxr_v], s_v, sem_s)
            cpt = pltpu.async_copy(x_hbm.at[idxc_v], t_v, sem_t)
            cps.wait()
            pltpu.sync_copy(s_v, gs_hbm.at[pl.ds(off, 128)])
            cpt.wait()
            pltpu.sync_copy(t_v, gt_hbm.at[pl.ds(off, 128)])
            return carry

        lax.fori_loop(0, WROWS, body, 0)

    return gather_k(x, row_p, col_p)


def _sc_segment_sum(eout_p, col2d):
    mesh = plsc.VectorSubcoreMesh(core_axis_name="c", subcore_axis_name="s",
                                  num_cores=NC, num_subcores=NS)
    CR = 8

    @functools.partial(
        pl.kernel,
        out_type=jax.ShapeDtypeStruct((NC, NPAD, OE), jnp.float32),
        mesh=mesh,
        scratch_types=[
            pltpu.VMEM((CR, 128), jnp.int32),
            pltpu.VMEM((CR * 128, OE), jnp.float32),
            pltpu.VMEM((STRIPE, OE), jnp.float32),
            pltpu.VMEM_SHARED((NPAD, OE), jnp.float32),
        ],
        compiler_params=pltpu.CompilerParams(use_tc_tiling_on_sc=False),
    )
    def scatter_k(eout_hbm, col_hbm, out_hbm, idx_v, rows_v, zb_v, acc_sp):
        cid = lax.axis_index("c")
        sid = lax.axis_index("s")
        wid = sid * NC + cid

        def zbody(i, carry):
            zb_v[i] = jnp.zeros((OE,), jnp.float32)
            return carry

        lax.fori_loop(0, STRIPE, zbody, 0)
        pltpu.sync_copy(zb_v, acc_sp.at[pl.ds(sid * STRIPE, STRIPE)])
        plsc.subcore_barrier()

        base_row = wid * WROWS

        def body(j, carry):
            r0 = base_row + j * CR
            pltpu.sync_copy(col_hbm.at[pl.ds(r0, CR)], idx_v)
            pltpu.sync_copy(eout_hbm.at[pl.ds(r0 * 128, CR * 128)], rows_v)

            def inner(k, c2):
                pltpu.sync_copy(rows_v.at[pl.ds(k * 128, 128)],
                                acc_sp.at[idx_v.at[k]], add=True)
                return c2

            lax.fori_loop(0, CR, inner, 0)
            return carry

        lax.fori_loop(0, WROWS // CR, body, 0)
        plsc.subcore_barrier()

        pltpu.sync_copy(acc_sp.at[pl.ds(sid * STRIPE, STRIPE)], zb_v)
        pltpu.sync_copy(zb_v, out_hbm.at[cid].at[pl.ds(sid * STRIPE, STRIPE)])

    return scatter_k(eout_p, col2d)


def _edge_body(gs_ref, gt_ref, ef_ref, A_ref, Bm_ref, Wef_ref, wsq_ref,
               wdist_ref, eb1_ref, eg1_ref, ebt1_ref, eW2_ref, eb2_ref,
               eg2_ref, ebt2_ref, out_ref):
    s = gs_ref[...]
    t = gt_ref[...]
    diff = s - t
    sq = jnp.sum(diff * diff, axis=1, keepdims=True)
    dist = jnp.sqrt(sq + 1e-12)
    h = (jnp.dot(s, A_ref[...], preferred_element_type=jnp.float32)
         + jnp.dot(t, Bm_ref[...], preferred_element_type=jnp.float32)
         + jnp.dot(ef_ref[...], Wef_ref[...], preferred_element_type=jnp.float32)
         + sq * wsq_ref[...] + dist * wdist_ref[...] + eb1_ref[...])
    h = _ln_gelu(h, eg1_ref[...], ebt1_ref[...])
    h2 = jnp.dot(h, eW2_ref[...], preferred_element_type=jnp.float32) + eb2_ref[...]
    h2 = _ln_gelu(h2, eg2_ref[...], ebt2_ref[...])
    gid = pl.program_id(0) * BE + lax.broadcasted_iota(jnp.int32, (BE, 1), 0)
    out_ref[...] = jnp.where(gid < E, h2, 0.0)


def _tc_edge(gs, gt, ef_p, A, Bm, Wef, wsq, wdist, eb1, eg1, ebt1,
             eW2, eb2, eg2, ebt2):
    full = lambda shape: pl.BlockSpec(shape, lambda i: (0, 0))
    return pl.pallas_call(
        _edge_body,
        grid=(EPAD // BE,),
        in_specs=[
            pl.BlockSpec((BE, D), lambda i: (i, 0)),
            pl.BlockSpec((BE, D), lambda i: (i, 0)),
            pl.BlockSpec((BE, DE), lambda i: (i, 0)),
            full((D, EH)), full((D, EH)), full((DE, EH)),
            full((1, EH)), full((1, EH)), full((1, EH)), full((1, EH)),
            full((1, EH)),
            full((EH, OE)), full((1, OE)), full((1, OE)), full((1, OE)),
        ],
        out_specs=pl.BlockSpec((BE, OE), lambda i: (i, 0)),
        out_shape=jax.ShapeDtypeStruct((EPAD, OE), jnp.float32),
    )(gs, gt, ef_p, A, Bm, Wef, wsq, wdist, eb1, eg1, ebt1,
      eW2, eb2, eg2, ebt2)


def _node_body(x_ref, ap_ref, W1x_ref, W1a_ref, nb1_ref, ng1_ref, nbt1_ref,
               nW2_ref, nb2_ref, ng2_ref, nbt2_ref, out_ref):
    x = x_ref[...]
    a = ap_ref[0] + ap_ref[1]
    h = (jnp.dot(x, W1x_ref[...], preferred_element_type=jnp.float32)
         + jnp.dot(a, W1a_ref[...], preferred_element_type=jnp.float32)
         + nb1_ref[...])
    h = _ln_gelu(h, ng1_ref[...], nbt1_ref[...])
    o = jnp.dot(h, nW2_ref[...], preferred_element_type=jnp.float32) + nb2_ref[...]
    out_ref[...] = _ln_gelu(o, ng2_ref[...], nbt2_ref[...])


def _tc_node(x, ap, W1x, W1a, nb1, ng1, nbt1, nW2, nb2, ng2, nbt2):
    full = lambda shape: pl.BlockSpec(shape, lambda i: (0, 0))
    full3 = lambda shape: pl.BlockSpec(shape, lambda i: (0, i, 0))
    return pl.pallas_call(
        _node_body,
        grid=(N // BN,),
        in_specs=[
            pl.BlockSpec((BN, D), lambda i: (i, 0)),
            full3((NC, BN, OE)),
            full((D, NH)), full((OE, NH)),
            full((1, NH)), full((1, NH)), full((1, NH)),
            full((NH, ON)), full((1, ON)), full((1, ON)), full((1, ON)),
        ],
        out_specs=pl.BlockSpec((BN, ON), lambda i: (i, 0)),
        out_shape=jax.ShapeDtypeStruct((N, ON), jnp.float32),
    )(x, ap, W1x, W1a, nb1, ng1, nbt1, nW2, nb2, ng2, nbt2)


def kernel(node_features, edge_index, edge_features,
           eW1, eb1, eg1, ebt1, eW2, eb2, eg2, ebt2,
           nW1, nb1, ng1, nbt1, nW2, nb2, ng2, nbt2):
    row = edge_index[0].astype(jnp.int32)
    col = edge_index[1].astype(jnp.int32)
    row_p = jnp.pad(row, (0, EPAD - E))
    col_p = jnp.pad(col, (0, EPAD - E))
    ef_p = jnp.pad(edge_features, ((0, EPAD - E), (0, 0)))

    A = eW1[0:D] + eW1[2 * D:3 * D]
    Bm = eW1[D:2 * D] - eW1[2 * D:3 * D]
    wsq = eW1[3 * D:3 * D + 1]
    wdist = eW1[3 * D + 1:3 * D + 2]
    Wef = eW1[3 * D + 2:]

    r2 = lambda v: v.reshape(1, -1)

    gs, gt = _sc_gather(node_features, row_p, col_p)
    eout_p = _tc_edge(gs, gt, ef_p, A, Bm, Wef, wsq, wdist,
                      r2(eb1), r2(eg1), r2(ebt1), eW2, r2(eb2), r2(eg2),
                      r2(ebt2))
    e_out = eout_p[:E]

    col2d = col_p.reshape(EPAD // 128, 128)
    partials = _sc_segment_sum(eout_p, col2d)

    W1x = nW1[0:D]
    W1a = nW1[D:]
    n_out = _tc_node(node_features, partials, W1x, W1a,
                     r2(nb1), r2(ng1), r2(nbt1),
                     nW2, r2(nb2), r2(ng2), r2(nbt2))
    return (n_out, e_out)


# speedup vs baseline: 1.4812x; 1.2003x over previous
import functools

import jax
import jax.numpy as jnp
from jax import lax
from jax.experimental import pallas as pl
from jax.experimental.pallas import tpu as pltpu
from jax.experimental.pallas import tpu_sc as plsc

N = 10000
E = 320000
D = 128
DE = 16
EH = 128
OE = 16
NH = 128
ON = 128

NC = 2
NS = 16
NW = NC * NS

WROWS = 80
EPAD = NW * WROWS * 128

NPAD = 10240
STRIPE = NPAD // NS

BE = 4096
BN = 2000

_SQRT_HALF = 0.7071067811865476


def _ln_gelu(x, g, b):
    mu = jnp.mean(x, axis=-1, keepdims=True)
    xc = x - mu
    var = jnp.mean(xc * xc, axis=-1, keepdims=True)
    y = xc * lax.rsqrt(var + 1e-5) * g + b
    return 0.5 * y * (1.0 + lax.erf(y * _SQRT_HALF))


def _sc_gather(x, row_p, col_p, half):
    mesh = plsc.VectorSubcoreMesh(core_axis_name="c", subcore_axis_name="s",
                                  num_cores=NC, num_subcores=NS)

    @functools.partial(
        pl.kernel,
        out_type=[
            jax.ShapeDtypeStruct((EPAD // 2, D), jnp.float32),
            jax.ShapeDtypeStruct((EPAD // 2, D), jnp.float32),
        ],
        mesh=mesh,
        scratch_types=[
            pltpu.VMEM((128,), jnp.int32),
            pltpu.VMEM((128,), jnp.int32),
            pltpu.VMEM((128, D), jnp.float32),
            pltpu.VMEM((128, D), jnp.float32),
            pltpu.SemaphoreType.DMA,
            pltpu.SemaphoreType.DMA,
        ],
        compiler_params=pltpu.CompilerParams(use_tc_tiling_on_sc=False),
    )
    def gather_k(x_hbm, row_hbm, col_hbm, gs_hbm, gt_hbm,
                 idxr_v, idxc_v, s_v, t_v, sem_s, sem_t):
        wid = lax.axis_index("s") * NC + lax.axis_index("c")
        base = wid * (WROWS // 2 * 128)

        def body(j, carry):
            off = base + j * 128
            pltpu.sync_copy(row_hbm.at[pl.ds(half * (EPAD // 2) + off, 128)],
                            idxr_v)
            pltpu.sync_copy(col_hbm.at[pl.ds(half * (EPAD // 2) + off, 128)],
                            idxc_v)
            cps = pltpu.async_copy(x_hbm.at[idxr_v], s_v, sem_s)
            cpt = pltpu.async_copy(x_hbm.at[idxc_v], t_v, sem_t)
            cps.wait()
            pltpu.sync_copy(s_v, gs_hbm.at[pl.ds(off, 128)])
            cpt.wait()
            pltpu.sync_copy(t_v, gt_hbm.at[pl.ds(off, 128)])
            return carry

        lax.fori_loop(0, WROWS // 2, body, 0)

    return gather_k(x, row_p, col_p)


def _sc_segment_sum(eout_p, col2d):
    mesh = plsc.VectorSubcoreMesh(core_axis_name="c", subcore_axis_name="s",
                                  num_cores=NC, num_subcores=NS)
    CR = 8

    @functools.partial(
        pl.kernel,
        out_type=jax.ShapeDtypeStruct((NC, NPAD, OE), jnp.float32),
        mesh=mesh,
        scratch_types=[
            pltpu.VMEM((CR, 128), jnp.int32),
            pltpu.VMEM((CR * 128, OE), jnp.float32),
            pltpu.VMEM((STRIPE, OE), jnp.float32),
            pltpu.VMEM_SHARED((NPAD, OE), jnp.float32),
        ],
        compiler_params=pltpu.CompilerParams(use_tc_tiling_on_sc=False),
    )
    def scatter_k(eout1_hbm, eout2_hbm, col_hbm, out_hbm, idx_v, rows_v,
                  zb_v, acc_sp):
        cid = lax.axis_index("c")
        sid = lax.axis_index("s")
        wid = sid * NC + cid

        def zbody(i, carry):
            zb_v[i] = jnp.zeros((OE,), jnp.float32)
            return carry

        lax.fori_loop(0, STRIPE, zbody, 0)
        pltpu.sync_copy(zb_v, acc_sp.at[pl.ds(sid * STRIPE, STRIPE)])
        plsc.subcore_barrier()

        base_row = wid * WROWS

        hrows = EPAD // 2 // 128

        def body(j, carry):
            r0 = base_row + j * CR
            pltpu.sync_copy(col_hbm.at[pl.ds(r0, CR)], idx_v)

            @pl.when(r0 < hrows)
            def _():
                pltpu.sync_copy(eout1_hbm.at[pl.ds(r0 * 128, CR * 128)],
                                rows_v)

            @pl.when(r0 >= hrows)
            def _():
                pltpu.sync_copy(
                    eout2_hbm.at[pl.ds((r0 - hrows) * 128, CR * 128)], rows_v)

            def inner(k, c2):
                pltpu.sync_copy(rows_v.at[pl.ds(k * 128, 128)],
                                acc_sp.at[idx_v.at[k]], add=True)
                return c2

            lax.fori_loop(0, CR, inner, 0)
            return carry

        lax.fori_loop(0, WROWS // CR, body, 0)
        plsc.subcore_barrier()

        pltpu.sync_copy(acc_sp.at[pl.ds(sid * STRIPE, STRIPE)], zb_v)
        pltpu.sync_copy(zb_v, out_hbm.at[cid].at[pl.ds(sid * STRIPE, STRIPE)])

    return scatter_k(eout_p[0], eout_p[1], col2d)


def _edge_body(hoff_ref, gs_ref, gt_ref, ef_ref, A_ref, Bm_ref, Wef_ref,
               wsq_ref, wdist_ref, eb1_ref, eg1_ref, ebt1_ref, eW2_ref,
               eb2_ref, eg2_ref, ebt2_ref, out_ref):
    s = gs_ref[...]
    t = gt_ref[...]
    diff = s - t
    sq = jnp.sum(diff * diff, axis=1, keepdims=True)
    dist = jnp.sqrt(sq + 1e-12)
    h = (jnp.dot(s, A_ref[...], preferred_element_type=jnp.float32)
         + jnp.dot(t, Bm_ref[...], preferred_element_type=jnp.float32)
         + jnp.dot(ef_ref[...], Wef_ref[...], preferred_element_type=jnp.float32)
         + sq * wsq_ref[...] + dist * wdist_ref[...] + eb1_ref[...])
    h = _ln_gelu(h, eg1_ref[...], ebt1_ref[...])
    h2 = jnp.dot(h, eW2_ref[...], preferred_element_type=jnp.float32) + eb2_ref[...]
    h2 = _ln_gelu(h2, eg2_ref[...], ebt2_ref[...])
    gid = (hoff_ref[0] + pl.program_id(0) * BE
           + lax.broadcasted_iota(jnp.int32, (BE, 1), 0))
    out_ref[...] = jnp.where(gid < E, h2, 0.0)


def _tc_edge(hoff, gs, gt, ef_p, A, Bm, Wef, wsq, wdist, eb1, eg1, ebt1,
             eW2, eb2, eg2, ebt2):
    full = lambda shape: pl.BlockSpec(shape, lambda i: (0, 0))
    return pl.pallas_call(
        _edge_body,
        grid=(EPAD // 2 // BE,),
        in_specs=[
            pl.BlockSpec(memory_space=pltpu.SMEM),
            pl.BlockSpec((BE, D), lambda i: (i, 0)),
            pl.BlockSpec((BE, D), lambda i: (i, 0)),
            pl.BlockSpec((BE, DE), lambda i: (i, 0)),
            full((D, EH)), full((D, EH)), full((DE, EH)),
            full((1, EH)), full((1, EH)), full((1, EH)), full((1, EH)),
            full((1, EH)),
            full((EH, OE)), full((1, OE)), full((1, OE)), full((1, OE)),
        ],
        out_specs=pl.BlockSpec((BE, OE), lambda i: (i, 0)),
        out_shape=jax.ShapeDtypeStruct((EPAD // 2, OE), jnp.float32),
    )(hoff, gs, gt, ef_p, A, Bm, Wef, wsq, wdist, eb1, eg1, ebt1,
      eW2, eb2, eg2, ebt2)


def _node_body(x_ref, ap_ref, W1x_ref, W1a_ref, nb1_ref, ng1_ref, nbt1_ref,
               nW2_ref, nb2_ref, ng2_ref, nbt2_ref, out_ref):
    x = x_ref[...]
    a = ap_ref[0] + ap_ref[1]
    h = (jnp.dot(x, W1x_ref[...], preferred_element_type=jnp.float32)
         + jnp.dot(a, W1a_ref[...], preferred_element_type=jnp.float32)
         + nb1_ref[...])
    h = _ln_gelu(h, ng1_ref[...], nbt1_ref[...])
    o = jnp.dot(h, nW2_ref[...], preferred_element_type=jnp.float32) + nb2_ref[...]
    out_ref[...] = _ln_gelu(o, ng2_ref[...], nbt2_ref[...])


def _tc_node(x, ap, W1x, W1a, nb1, ng1, nbt1, nW2, nb2, ng2, nbt2):
    full = lambda shape: pl.BlockSpec(shape, lambda i: (0, 0))
    full3 = lambda shape: pl.BlockSpec(shape, lambda i: (0, i, 0))
    return pl.pallas_call(
        _node_body,
        grid=(N // BN,),
        in_specs=[
            pl.BlockSpec((BN, D), lambda i: (i, 0)),
            full3((NC, BN, OE)),
            full((D, NH)), full((OE, NH)),
            full((1, NH)), full((1, NH)), full((1, NH)),
            full((NH, ON)), full((1, ON)), full((1, ON)), full((1, ON)),
        ],
        out_specs=pl.BlockSpec((BN, ON), lambda i: (i, 0)),
        out_shape=jax.ShapeDtypeStruct((N, ON), jnp.float32),
    )(x, ap, W1x, W1a, nb1, ng1, nbt1, nW2, nb2, ng2, nbt2)


def kernel(node_features, edge_index, edge_features,
           eW1, eb1, eg1, ebt1, eW2, eb2, eg2, ebt2,
           nW1, nb1, ng1, nbt1, nW2, nb2, ng2, nbt2):
    row = edge_index[0].astype(jnp.int32)
    col = edge_index[1].astype(jnp.int32)
    row_p = jnp.pad(row, (0, EPAD - E))
    col_p = jnp.pad(col, (0, EPAD - E))
    ef_p = jnp.pad(edge_features, ((0, EPAD - E), (0, 0)))

    A = eW1[0:D] + eW1[2 * D:3 * D]
    Bm = eW1[D:2 * D] - eW1[2 * D:3 * D]
    wsq = eW1[3 * D:3 * D + 1]
    wdist = eW1[3 * D + 1:3 * D + 2]
    Wef = eW1[3 * D + 2:]

    r2 = lambda v: v.reshape(1, -1)

    H = EPAD // 2
    ef1 = ef_p[:H]
    ef2 = ef_p[H:]
    gs1, gt1 = _sc_gather(node_features, row_p, col_p, 0)
    gs2, gt2 = _sc_gather(node_features, row_p, col_p, 1)
    eargs = (A, Bm, Wef, wsq, wdist, r2(eb1), r2(eg1), r2(ebt1), eW2,
             r2(eb2), r2(eg2), r2(ebt2))
    e1 = _tc_edge(jnp.zeros((1,), jnp.int32), gs1, gt1, ef1, *eargs)
    e2 = _tc_edge(jnp.full((1,), H, jnp.int32), gs2, gt2, ef2, *eargs)
    e_out = jnp.concatenate([e1, e2], axis=0)[:E]

    col2d = col_p.reshape(EPAD // 128, 128)
    partials = _sc_segment_sum((e1, e2), col2d)

    W1x = nW1[0:D]
    W1a = nW1[D:]
    n_out = _tc_node(node_features, partials, W1x, W1a,
                     r2(nb1), r2(ng1), r2(nbt1),
                     nW2, r2(nb2), r2(ng2), r2(nbt2))
    return (n_out, e_out)


# speedup vs baseline: 1.5735x; 1.0623x over previous
import functools

import jax
import jax.numpy as jnp
from jax import lax
from jax.experimental import pallas as pl
from jax.experimental.pallas import tpu as pltpu
from jax.experimental.pallas import tpu_sc as plsc

N = 10000
E = 320000
D = 128
DE = 16
EH = 128
OE = 16
NH = 128
ON = 128

NC = 2
NS = 16
NW = NC * NS

WROWS = 80
EPAD = NW * WROWS * 128

NPAD = 10240
STRIPE = NPAD // NS

BE = 4096
BN = 2000

_SQRT_HALF = 0.7071067811865476


def _ln_gelu(x, g, b):
    mu = jnp.mean(x, axis=-1, keepdims=True)
    xc = x - mu
    var = jnp.mean(xc * xc, axis=-1, keepdims=True)
    y = xc * lax.rsqrt(var + 1e-5) * g + b
    return 0.5 * y * (1.0 + lax.erf(y * _SQRT_HALF))


def _sc_gather(x, row_p, col_p, half):
    mesh = plsc.VectorSubcoreMesh(core_axis_name="c", subcore_axis_name="s",
                                  num_cores=NC, num_subcores=NS)

    @functools.partial(
        pl.kernel,
        out_type=[
            jax.ShapeDtypeStruct((EPAD // 4, D), jnp.float32),
            jax.ShapeDtypeStruct((EPAD // 4, D), jnp.float32),
        ],
        mesh=mesh,
        scratch_types=[
            pltpu.VMEM((128,), jnp.int32),
            pltpu.VMEM((128,), jnp.int32),
            pltpu.VMEM((128, D), jnp.float32),
            pltpu.VMEM((128, D), jnp.float32),
            pltpu.SemaphoreType.DMA,
            pltpu.SemaphoreType.DMA,
        ],
        compiler_params=pltpu.CompilerParams(use_tc_tiling_on_sc=False),
    )
    def gather_k(x_hbm, row_hbm, col_hbm, gs_hbm, gt_hbm,
                 idxr_v, idxc_v, s_v, t_v, sem_s, sem_t):
        wid = lax.axis_index("s") * NC + lax.axis_index("c")
        base = wid * (WROWS // 4 * 128)

        def body(j, carry):
            off = base + j * 128
            pltpu.sync_copy(row_hbm.at[pl.ds(half * (EPAD // 4) + off, 128)],
                            idxr_v)
            pltpu.sync_copy(col_hbm.at[pl.ds(half * (EPAD // 4) + off, 128)],
                            idxc_v)
            cps = pltpu.async_copy(x_hbm.at[idxr_v], s_v, sem_s)
            cpt = pltpu.async_copy(x_hbm.at[idxc_v], t_v, sem_t)
            cps.wait()
            pltpu.sync_copy(s_v, gs_hbm.at[pl.ds(off, 128)])
            cpt.wait()
            pltpu.sync_copy(t_v, gt_hbm.at[pl.ds(off, 128)])
            return carry

        lax.fori_loop(0, WROWS // 4, body, 0)

    return gather_k(x, row_p, col_p)


def _sc_segment_sum(eout_p, col2d):
    mesh = plsc.VectorSubcoreMesh(core_axis_name="c", subcore_axis_name="s",
                                  num_cores=NC, num_subcores=NS)
    CR = 8

    @functools.partial(
        pl.kernel,
        out_type=jax.ShapeDtypeStruct((NC, NPAD, OE), jnp.float32),
        mesh=mesh,
        scratch_types=[
            pltpu.VMEM((CR, 128), jnp.int32),
            pltpu.VMEM((CR * 128, OE), jnp.float32),
            pltpu.VMEM((STRIPE, OE), jnp.float32),
            pltpu.VMEM_SHARED((NPAD, OE), jnp.float32),
        ],
        compiler_params=pltpu.CompilerParams(use_tc_tiling_on_sc=False),
    )
    def scatter_k(eout1_hbm, eout2_hbm, eout3_hbm, eout4_hbm, col_hbm,
                  out_hbm, idx_v, rows_v, zb_v, acc_sp):
        cid = lax.axis_index("c")
        sid = lax.axis_index("s")
        wid = sid * NC + cid

        def zbody(i, carry):
            zb_v[i] = jnp.zeros((OE,), jnp.float32)
            return carry

        lax.fori_loop(0, STRIPE, zbody, 0)
        pltpu.sync_copy(zb_v, acc_sp.at[pl.ds(sid * STRIPE, STRIPE)])
        plsc.subcore_barrier()

        base_row = wid * WROWS

        qrows = EPAD // 4 // 128

        def body(j, carry):
            r0 = base_row + j * CR
            pltpu.sync_copy(col_hbm.at[pl.ds(r0, CR)], idx_v)
            for qi, src_hbm in enumerate((eout1_hbm, eout2_hbm, eout3_hbm,
                                          eout4_hbm)):
                @pl.when((r0 >= qi * qrows) & (r0 < (qi + 1) * qrows))
                def _(qi=qi, src_hbm=src_hbm):
                    pltpu.sync_copy(
                        src_hbm.at[pl.ds((r0 - qi * qrows) * 128, CR * 128)],
                        rows_v)

            def inner(k, c2):
                pltpu.sync_copy(rows_v.at[pl.ds(k * 128, 128)],
                                acc_sp.at[idx_v.at[k]], add=True)
                return c2

            lax.fori_loop(0, CR, inner, 0)
            return carry

        lax.fori_loop(0, WROWS // CR, body, 0)
        plsc.subcore_barrier()

        pltpu.sync_copy(acc_sp.at[pl.ds(sid * STRIPE, STRIPE)], zb_v)
        pltpu.sync_copy(zb_v, out_hbm.at[cid].at[pl.ds(sid * STRIPE, STRIPE)])

    return scatter_k(eout_p[0], eout_p[1], eout_p[2], eout_p[3], col2d)


def _edge_body(hoff_ref, gs_ref, gt_ref, ef_ref, A_ref, Bm_ref, Wef_ref,
               wsq_ref, wdist_ref, eb1_ref, eg1_ref, ebt1_ref, eW2_ref,
               eb2_ref, eg2_ref, ebt2_ref, out_ref):
    s = gs_ref[...]
    t = gt_ref[...]
    diff = s - t
    sq = jnp.sum(diff * diff, axis=1, keepdims=True)
    dist = jnp.sqrt(sq + 1e-12)
    h = (jnp.dot(s, A_ref[...], preferred_element_type=jnp.float32)
         + jnp.dot(t, Bm_ref[...], preferred_element_type=jnp.float32)
         + jnp.dot(ef_ref[...], Wef_ref[...], preferred_element_type=jnp.float32)
         + sq * wsq_ref[...] + dist * wdist_ref[...] + eb1_ref[...])
    h = _ln_gelu(h, eg1_ref[...], ebt1_ref[...])
    h2 = jnp.dot(h, eW2_ref[...], preferred_element_type=jnp.float32) + eb2_ref[...]
    h2 = _ln_gelu(h2, eg2_ref[...], ebt2_ref[...])
    gid = (hoff_ref[0] + pl.program_id(0) * BE
           + lax.broadcasted_iota(jnp.int32, (BE, 1), 0))
    out_ref[...] = jnp.where(gid < E, h2, 0.0)


def _tc_edge(hoff, gs, gt, ef_p, A, Bm, Wef, wsq, wdist, eb1, eg1, ebt1,
             eW2, eb2, eg2, ebt2):
    full = lambda shape: pl.BlockSpec(shape, lambda i: (0, 0))
    return pl.pallas_call(
        _edge_body,
        grid=(EPAD // 4 // BE,),
        in_specs=[
            pl.BlockSpec(memory_space=pltpu.SMEM),
            pl.BlockSpec((BE, D), lambda i: (i, 0)),
            pl.BlockSpec((BE, D), lambda i: (i, 0)),
            pl.BlockSpec((BE, DE), lambda i: (i, 0)),
            full((D, EH)), full((D, EH)), full((DE, EH)),
            full((1, EH)), full((1, EH)), full((1, EH)), full((1, EH)),
            full((1, EH)),
            full((EH, OE)), full((1, OE)), full((1, OE)), full((1, OE)),
        ],
        out_specs=pl.BlockSpec((BE, OE), lambda i: (i, 0)),
        out_shape=jax.ShapeDtypeStruct((EPAD // 4, OE), jnp.float32),
    )(hoff, gs, gt, ef_p, A, Bm, Wef, wsq, wdist, eb1, eg1, ebt1,
      eW2, eb2, eg2, ebt2)


def _node_body(x_ref, ap_ref, W1x_ref, W1a_ref, nb1_ref, ng1_ref, nbt1_ref,
               nW2_ref, nb2_ref, ng2_ref, nbt2_ref, out_ref):
    x = x_ref[...]
    a = ap_ref[0] + ap_ref[1]
    h = (jnp.dot(x, W1x_ref[...], preferred_element_type=jnp.float32)
         + jnp.dot(a, W1a_ref[...], preferred_element_type=jnp.float32)
         + nb1_ref[...])
    h = _ln_gelu(h, ng1_ref[...], nbt1_ref[...])
    o = jnp.dot(h, nW2_ref[...], preferred_element_type=jnp.float32) + nb2_ref[...]
    out_ref[...] = _ln_gelu(o, ng2_ref[...], nbt2_ref[...])


def _tc_node(x, ap, W1x, W1a, nb1, ng1, nbt1, nW2, nb2, ng2, nbt2):
    full = lambda shape: pl.BlockSpec(shape, lambda i: (0, 0))
    full3 = lambda shape: pl.BlockSpec(shape, lambda i: (0, i, 0))
    return pl.pallas_call(
        _node_body,
        grid=(N // BN,),
        in_specs=[
            pl.BlockSpec((BN, D), lambda i: (i, 0)),
            full3((NC, BN, OE)),
            full((D, NH)), full((OE, NH)),
            full((1, NH)), full((1, NH)), full((1, NH)),
            full((NH, ON)), full((1, ON)), full((1, ON)), full((1, ON)),
        ],
        out_specs=pl.BlockSpec((BN, ON), lambda i: (i, 0)),
        out_shape=jax.ShapeDtypeStruct((N, ON), jnp.float32),
    )(x, ap, W1x, W1a, nb1, ng1, nbt1, nW2, nb2, ng2, nbt2)


def kernel(node_features, edge_index, edge_features,
           eW1, eb1, eg1, ebt1, eW2, eb2, eg2, ebt2,
           nW1, nb1, ng1, nbt1, nW2, nb2, ng2, nbt2):
    row = edge_index[0].astype(jnp.int32)
    col = edge_index[1].astype(jnp.int32)
    row_p = jnp.pad(row, (0, EPAD - E))
    col_p = jnp.pad(col, (0, EPAD - E))
    ef_p = jnp.pad(edge_features, ((0, EPAD - E), (0, 0)))

    A = eW1[0:D] + eW1[2 * D:3 * D]
    Bm = eW1[D:2 * D] - eW1[2 * D:3 * D]
    wsq = eW1[3 * D:3 * D + 1]
    wdist = eW1[3 * D + 1:3 * D + 2]
    Wef = eW1[3 * D + 2:]

    r2 = lambda v: v.reshape(1, -1)

    Q = EPAD // 4
    eargs = (A, Bm, Wef, wsq, wdist, r2(eb1), r2(eg1), r2(ebt1), eW2,
             r2(eb2), r2(eg2), r2(ebt2))
    eq = []
    for q in range(4):
        gsq, gtq = _sc_gather(node_features, row_p, col_p, q)
        eq.append(_tc_edge(jnp.full((1,), q * Q, jnp.int32), gsq, gtq,
                           ef_p[q * Q:(q + 1) * Q], *eargs))
    e_out = jnp.concatenate(eq, axis=0)[:E]

    col2d = col_p.reshape(EPAD // 128, 128)
    partials = _sc_segment_sum(tuple(eq), col2d)

    W1x = nW1[0:D]
    W1a = nW1[D:]
    n_out = _tc_node(node_features, partials, W1x, W1a,
                     r2(nb1), r2(ng1), r2(nbt1),
                     nW2, r2(nb2), r2(ng2), r2(nbt2))
    return (n_out, e_out)
